# baseline (speedup 1.0000x reference)
import functools

import jax
import jax.numpy as jnp
from jax import lax
from jax.experimental import pallas as pl
from jax.experimental.pallas import tpu as pltpu
from jax.experimental.pallas import tpu_sc as plsc

N = 10000
E = 160000
D = 128
HEADS = 8
DH = 16
M = 64
NL = 6

NP = 10112
DN = N
NW = 32
K = 32
NB = 160
EW = NB * K
E2 = NW * EW
RPS = NP // 16

_EPS = 1e-5


def _ln(x, g, b):
    m = x.mean(-1, keepdims=True)
    d = x - m
    v = (d * d).mean(-1, keepdims=True)
    return d / jnp.sqrt(v + _EPS) * g + b


def _enc_body(x_ref, rwse_ref, a2d_ref,
              wn_ref, bn_ref, gn_ref, zn_ref,
              wrT_ref, br_ref, gr_ref, zr_ref,
              wpT_ref, bp_ref, gp_ref, zp_ref,
              we_ref, be_ref, ge_ref, ze_ref,
              w1iT_ref, b1_ref, w1jT_ref,
              h0_ref, a0_ref, b0_ref, uece_ref):
    x = x_ref[...]
    am = jnp.mean(x)
    dx = x - am
    av = jnp.mean(dx * dx)
    w = wn_ref[...]
    inv = gn_ref[...] / jnp.sqrt(av * w * w + _EPS)
    h = jnp.maximum(dx * (w * inv) + zn_ref[...], 0.0)
    z = jnp.dot(rwse_ref[...], wrT_ref[...],
                preferred_element_type=jnp.float32) + br_ref[...]
    zm = z.mean(0, keepdims=True)
    dz = z - zm
    zv = (dz * dz).mean(0, keepdims=True)
    h = h + jnp.maximum(dz / jnp.sqrt(zv + _EPS) * gr_ref[...] + zr_ref[...], 0.0)
    z = jnp.maximum(jnp.dot(h, wpT_ref[...],
                            preferred_element_type=jnp.float32) + bp_ref[...], 0.0)
    zm = z.mean(0, keepdims=True)
    dz = z - zm
    zv = (dz * dz).mean(0, keepdims=True)
    h0 = dz / jnp.sqrt(zv + _EPS) * gp_ref[...] + zp_ref[...]
    h0_ref[...] = h0
    a2 = a2d_ref[...]
    em = jnp.mean(a2)
    de = a2 - em
    ev = jnp.mean(de * de)
    we = we_ref[...]
    inve = ge_ref[...] / jnp.sqrt(ev * we * we + _EPS)
    uece_ref[0:1, :] = we * inve
    uece_ref[1:2, :] = (-em * we) * inve + ze_ref[...]
    a0_ref[pl.ds(0, N), :] = jnp.dot(h0, w1iT_ref[...],
                                     preferred_element_type=jnp.float32) + b1_ref[...]
    a0_ref[pl.ds(N, NP - N), :] = jnp.zeros((NP - N, D), jnp.float32)
    b0_ref[pl.ds(0, N), :] = jnp.dot(h0, w1jT_ref[...],
                                     preferred_element_type=jnp.float32)
    b0_ref[pl.ds(N, NP - N), :] = jnp.zeros((NP - N, D), jnp.float32)


def _enc_call(x, rwse, a2d, args, interpret=False):
    return pl.pallas_call(
        _enc_body,
        out_shape=[
            jax.ShapeDtypeStruct((N, D), jnp.float32),
            jax.ShapeDtypeStruct((NP, D), jnp.float32),
            jax.ShapeDtypeStruct((NP, D), jnp.float32),
            jax.ShapeDtypeStruct((2, D), jnp.float32),
        ],
        interpret=interpret,
    )(x, rwse, a2d, *args)


_CB = 2048


def _cmat_body(a_ref, uece_ref, weT_ref, *out_refs):
    a = a_ref[...]
    ue = uece_ref[0:1, :]
    ce = uece_ref[1:2, :]
    ea = jnp.maximum(a * ue + ce, 0.0)
    for l in range(NL):
        out_refs[l][...] = jnp.dot(ea, weT_ref[:, l * D:(l + 1) * D],
                                   preferred_element_type=jnp.float32)


def _cmat_call(a_pad, uece, weT_all, interpret=False):
    grid = E2 // _CB
    return pl.pallas_call(
        _cmat_body,
        grid=(grid,),
        in_specs=[
            pl.BlockSpec((_CB, 1), lambda i: (i, 0)),
            pl.BlockSpec((2, D), lambda i: (0, 0)),
            pl.BlockSpec((D, NL * D), lambda i: (0, 0)),
        ],
        out_specs=[pl.BlockSpec((_CB, D), lambda i: (i, 0)) for _ in range(NL)],
        out_shape=[jax.ShapeDtypeStruct((E2, D), jnp.float32) for _ in range(NL)],
        interpret=interpret,
    )(a_pad, uece, weT_all)


def _make_edge_kernel(interpret=False):
    mesh = plsc.VectorSubcoreMesh(core_axis_name="c", subcore_axis_name="s",
                                  num_cores=2, num_subcores=16)
    out_type = [jax.ShapeDtypeStruct((2, NP, D), jnp.float32)]
    scratch = [
        pltpu.VMEM((EW,), jnp.int32),
        pltpu.VMEM((EW,), jnp.int32),
        [pltpu.VMEM((K,), jnp.int32) for _ in range(2)],
        [pltpu.VMEM((K, D), jnp.float32) for _ in range(2)],
        [pltpu.VMEM((K, D), jnp.float32) for _ in range(2)],
        [pltpu.VMEM((K, D), jnp.float32) for _ in range(2)],
        [pltpu.VMEM((K, D), jnp.float32) for _ in range(2)],
        pltpu.VMEM_SHARED((NP, D), jnp.float32),
        [pltpu.SemaphoreType.DMA for _ in range(2)],
        [pltpu.SemaphoreType.DMA for _ in range(2)],
        [pltpu.SemaphoreType.DMA for _ in range(2)],
        [pltpu.SemaphoreType.DMA for _ in range(2)],
    ]

    def body(a_hbm, b_hbm, c_hbm, src_hbm, dst_hbm, out_s,
             src_v, dst_v, idx_sc, buf_a, buf_b, buf_c, buf_z, s_sh,
             sem_a, sem_b, sem_c, sem_sc):
        cid = lax.axis_index("c")
        sid = lax.axis_index("s")
        wid = sid * 2 + cid
        base0 = wid * EW

        pltpu.sync_copy(src_hbm.at[pl.ds(base0, EW)], src_v)
        pltpu.sync_copy(dst_hbm.at[pl.ds(base0, EW)], dst_v)

        def zrow(r, carry):
            for j in range(D // 16):
                buf_z[0][r, pl.ds(j * 16, 16)] = jnp.zeros((16,), jnp.float32)
            return carry
        lax.fori_loop(0, K, zrow, 0)
        r0 = sid * RPS
        off = 0
        while off < RPS:
            sz = min(K, RPS - off)
            pltpu.sync_copy(buf_z[0].at[pl.ds(0, sz)],
                            s_sh.at[pl.ds(r0 + off, sz)])
            off += sz
        plsc.subcore_barrier()

        def start_gathers(q, b):
            pltpu.async_copy(a_hbm.at[dst_v.at[pl.ds(b * K, K)]],
                             buf_a[q], sem_a[q])
            pltpu.async_copy(b_hbm.at[src_v.at[pl.ds(b * K, K)]],
                             buf_b[q], sem_b[q])
            pltpu.async_copy(c_hbm.at[pl.ds(base0 + b * K, K)],
                             buf_c[q], sem_c[q])

        def wait_gathers(q):
            pltpu.make_async_copy(a_hbm.at[dst_v.at[pl.ds(0, K)]],
                                  buf_a[q], sem_a[q]).wait()
            pltpu.make_async_copy(b_hbm.at[src_v.at[pl.ds(0, K)]],
                                  buf_b[q], sem_b[q]).wait()
            pltpu.make_async_copy(c_hbm.at[pl.ds(0, K)], buf_c[q], sem_c[q]).wait()

        for q in (0, 1):
            start_gathers(q, q)

        def step(i, carry):
            for q in (0, 1):
                b = 2 * i + q
                wait_gathers(q)

                @pl.when(i >= 1)
                def _drain():
                    pltpu.make_async_copy(
                        buf_z[q], s_sh.at[idx_sc[q]], sem_sc[q]).wait()

                def crow(r, carry2):
                    for j in range(D // 16):
                        s = pl.ds(j * 16, 16)
                        buf_z[q][r, s] = jnp.maximum(
                            buf_a[q][r, s] + buf_b[q][r, s] + buf_c[q][r, s],
                            0.0)
                    return carry2
                lax.fori_loop(0, K, crow, 0)
                for j in range(K // 16):
                    idx_sc[q][pl.ds(j * 16, 16)] = dst_v[pl.ds(b * K + j * 16, 16)]
                pltpu.async_copy(buf_z[q], s_sh.at[idx_sc[q]], sem_sc[q],
                                 add=True)

                @pl.when(b + 2 < NB)
                def _prefetch():
                    start_gathers(q, b + 2)
            return carry
        lax.fori_loop(0, NB // 2, step, 0)
        for q in (0, 1):
            pltpu.make_async_copy(buf_z[q], s_sh.at[idx_sc[q]], sem_sc[q]).wait()
        plsc.subcore_barrier()

        pltpu.sync_copy(s_sh.at[pl.ds(r0, RPS)], out_s.at[cid, pl.ds(r0, RPS)])

    return pl.kernel(body, out_type=out_type, mesh=mesh, scratch_types=scratch,
                     interpret=interpret)


def _make_deg_kernel(interpret=False):
    mesh = plsc.VectorSubcoreMesh(core_axis_name="c", subcore_axis_name="s",
                                  num_cores=2, num_subcores=16)
    out_type = [jax.ShapeDtypeStruct((2, NP, 16), jnp.float32)]
    scratch = [
        pltpu.VMEM((K,), jnp.int32),
        pltpu.VMEM((K, 16), jnp.float32),
        pltpu.VMEM((K, 16), jnp.float32),
        pltpu.VMEM_SHARED((NP, 16), jnp.float32),
        pltpu.SemaphoreType.DMA,
    ]

    def body(dst_hbm, out_d, idx_d, ones16, zer16, d_sh, sem):
        del sem
        cid = lax.axis_index("c")
        sid = lax.axis_index("s")
        wid = sid * 2 + cid

        def frow(r, carry):
            ones16[r, pl.ds(0, 16)] = jnp.ones((16,), jnp.float32)
            zer16[r, pl.ds(0, 16)] = jnp.zeros((16,), jnp.float32)
            return carry
        lax.fori_loop(0, K, frow, 0)
        r0 = sid * RPS
        off = 0
        while off < RPS:
            sz = min(K, RPS - off)
            pltpu.sync_copy(zer16.at[pl.ds(0, sz)], d_sh.at[pl.ds(r0 + off, sz)])
            off += sz
        plsc.subcore_barrier()

        def blk(b, carry):
            base = wid * EW + b * K
            pltpu.sync_copy(dst_hbm.at[pl.ds(base, K)], idx_d)
            pltpu.sync_copy(ones16, d_sh.at[idx_d], add=True)
            return carry
        lax.fori_loop(0, NB, blk, 0)
        plsc.subcore_barrier()
        pltpu.sync_copy(d_sh.at[pl.ds(r0, RPS)], out_d.at[cid, pl.ds(r0, RPS)])

    return pl.kernel(body, out_type=out_type, mesh=mesh, scratch_types=scratch,
                     interpret=interpret)


_DN_SCALE = DH ** -0.25
_RATIO = M ** -0.5


def _gcn_body(h_ref, s2_ref, dg2_ref,
              e2T_ref, b2_ref, n1hT_ref, n1aT_ref, bn1_ref, n2T_ref, bn2_ref,
              ggcn_ref, zgcn_ref, gloc_ref, zloc_ref, hl_ref):
    h = h_ref[...]
    su = s2_ref[0, pl.ds(0, N), :] + s2_ref[1, pl.ds(0, N), :]
    dgf = dg2_ref[0, pl.ds(0, N), :] + dg2_ref[1, pl.ds(0, N), :]
    dg = dgf[:, 0:1]
    agg = jnp.dot(su, e2T_ref[...], preferred_element_type=jnp.float32) \
        + dg * b2_ref[...]
    t = jnp.maximum(
        jnp.dot(h, n1hT_ref[...], preferred_element_type=jnp.float32)
        + jnp.dot(agg, n1aT_ref[...], preferred_element_type=jnp.float32)
        + bn1_ref[...], 0.0)
    og = jnp.dot(t, n2T_ref[...], preferred_element_type=jnp.float32) \
        + bn2_ref[...] + h
    hl = _ln(og, ggcn_ref[...], zgcn_ref[...])
    hl_ref[...] = _ln(h + hl, gloc_ref[...], zloc_ref[...])


def _gcn_call(h, s2, dg2, args, interpret=False):
    return pl.pallas_call(
        _gcn_body,
        out_shape=jax.ShapeDtypeStruct((N, D), jnp.float32),
        interpret=interpret,
    )(h, s2, dg2, *args)


def _kmax_body(h_ref, kT_ref, kb_ref, projT_ref, out_ref):
    h = h_ref[...]
    kT = kT_ref[...]
    kb = kb_ref[...]
    projT = projT_ref[...]
    vals = []
    for hh in range(HEADS):
        sl = slice(hh * DH, (hh + 1) * DH)
        kh = (jnp.dot(h, kT[:, sl], preferred_element_type=jnp.float32)
              + kb[:, sl]) * _DN_SCALE
        ukh = jnp.dot(kh, projT, preferred_element_type=jnp.float32)
        vals.append(jnp.max(ukh).reshape(1, 1))
    out_ref[...] = jnp.concatenate(vals, axis=1)


def _kmax_call(h, kT, kb, projT, interpret=False):
    return pl.pallas_call(
        _kmax_body,
        out_shape=jax.ShapeDtypeStruct((1, HEADS), jnp.float32),
        interpret=interpret,
    )(h, kT, kb, projT)


def _attn_body(h_ref, qTh_ref, qbh_ref, kTh_ref, kbh_ref, vTh_ref, vbh_ref,
               oTh_ref, ob_ref, projT_ref, kmax_ref, gat_ref, zat_ref,
               ha_ref, acc_ref):
    i = pl.program_id(0)
    h = h_ref[...]
    projT = projT_ref[...]
    mk = jnp.max(kmax_ref[...])
    qh = (jnp.dot(h, qTh_ref[0], preferred_element_type=jnp.float32)
          + qbh_ref[0]) * _DN_SCALE
    kh = (jnp.dot(h, kTh_ref[0], preferred_element_type=jnp.float32)
          + kbh_ref[0]) * _DN_SCALE
    vh = jnp.dot(h, vTh_ref[0], preferred_element_type=jnp.float32) + vbh_ref[0]
    uq = jnp.dot(qh, projT, preferred_element_type=jnp.float32)
    uk = jnp.dot(kh, projT, preferred_element_type=jnp.float32)
    d2q = 0.5 * jnp.sum(qh * qh, axis=1, keepdims=True)
    d2k = 0.5 * jnp.sum(kh * kh, axis=1, keepdims=True)
    mq = jnp.max(uq, axis=1, keepdims=True)
    qp = _RATIO * (jnp.exp(uq - d2q - mq) + 1e-4)
    kp = _RATIO * (jnp.exp(uk - d2k - mk) + 1e-4)
    vh1 = jnp.concatenate([vh, jnp.ones((N, 1), jnp.float32)], axis=1)
    ctx = lax.dot_general(kp, vh1, (((0,), (0,)), ((), ())),
                          preferred_element_type=jnp.float32)
    num = jnp.dot(qp, ctx, preferred_element_type=jnp.float32)
    o_h = num[:, :DH] / (num[:, DH:DH + 1] + 1e-6)
    contrib = jnp.dot(o_h, oTh_ref[0], preferred_element_type=jnp.float32)

    @pl.when(i == 0)
    def _init():
        acc_ref[...] = contrib

    @pl.when(i > 0)
    def _acc():
        acc_ref[...] = acc_ref[...] + contrib

    @pl.when(i == HEADS - 1)
    def _fin():
        attn = acc_ref[...] + ob_ref[...]
        ha_ref[...] = _ln(h + attn, gat_ref[...], zat_ref[...])


def _attn_call(h, qTh, qbh, kTh, kbh, vTh, vbh, oTh, ob, projT, kmax,
               gat, zat, interpret=False):
    full2 = lambda shape: pl.BlockSpec(shape, lambda i: (0, 0))
    headb = lambda shape: pl.BlockSpec(shape, lambda i: (i, 0, 0))
    return pl.pallas_call(
        _attn_body,
        grid=(HEADS,),
        in_specs=[
            full2((N, D)),
            headb((1, D, DH)), headb((1, 1, DH)),
            headb((1, D, DH)), headb((1, 1, DH)),
            headb((1, D, DH)), headb((1, 1, DH)),
            headb((1, DH, D)),
            full2((1, D)), full2((DH, M)), full2((1, HEADS)),
            full2((1, D)), full2((1, D)),
        ],
        out_specs=pl.BlockSpec((N, D), lambda i: (0, 0)),
        out_shape=jax.ShapeDtypeStruct((N, D), jnp.float32),
        scratch_shapes=[pltpu.VMEM((N, D), jnp.float32)],
        compiler_params=pltpu.CompilerParams(vmem_limit_bytes=100 * 1024 * 1024),
        interpret=interpret,
    )(h, qTh, qbh, kTh, kbh, vTh, vbh, oTh, ob, projT, kmax, gat, zat)


def _merge_body(has_next, *refs):
    (hl_ref, ha_ref, f1T_ref, f1b_ref, f2T_ref, f2b_ref,
     gff_ref, zff_ref) = refs[:8]
    if has_next:
        (w1iT_ref, b1n_ref, w1jT_ref) = refs[8:11]
        (hout_ref, an_ref, bn_ref) = refs[11:]
    else:
        (hout_ref,) = refs[8:]
    hn = hl_ref[...] + ha_ref[...]
    ff = jnp.dot(
        jnp.maximum(jnp.dot(hn, f1T_ref[...],
                            preferred_element_type=jnp.float32) + f1b_ref[...], 0.0),
        f2T_ref[...], preferred_element_type=jnp.float32) + f2b_ref[...]
    hout = _ln(hn + ff, gff_ref[...], zff_ref[...])
    hout_ref[...] = hout
    if has_next:
        an_ref[pl.ds(0, N), :] = jnp.dot(
            hout, w1iT_ref[...], preferred_element_type=jnp.float32) + b1n_ref[...]
        an_ref[pl.ds(N, NP - N), :] = jnp.zeros((NP - N, D), jnp.float32)
        bn_ref[pl.ds(0, N), :] = jnp.dot(
            hout, w1jT_ref[...], preferred_element_type=jnp.float32)
        bn_ref[pl.ds(N, NP - N), :] = jnp.zeros((NP - N, D), jnp.float32)


def _merge_call(hl, ha, args, has_next, interpret=False):
    out_shape = [jax.ShapeDtypeStruct((N, D), jnp.float32)]
    if has_next:
        out_shape.append(jax.ShapeDtypeStruct((NP, D), jnp.float32))
        out_shape.append(jax.ShapeDtypeStruct((NP, D), jnp.float32))
    return pl.pallas_call(
        functools.partial(_merge_body, has_next),
        out_shape=out_shape,
        interpret=interpret,
    )(hl, ha, *args)


def _head_body(h_ref, w1T_ref, b1_ref, w2T_ref, b2_ref, out_ref):
    t = jnp.maximum(jnp.dot(h_ref[...], w1T_ref[...],
                            preferred_element_type=jnp.float32) + b1_ref[...], 0.0)
    out_ref[...] = jnp.dot(t, w2T_ref[...],
                           preferred_element_type=jnp.float32) + b2_ref[...]


def _head_call(h, w1T, b1, w2T, b2, interpret=False):
    return pl.pallas_call(
        _head_body,
        out_shape=jax.ShapeDtypeStruct((N, 1), jnp.float32),
        interpret=interpret,
    )(h, w1T, b1, w2T, b2)


def _row(x):
    return x.reshape(1, -1).astype(jnp.float32)


def kernel(x, edge_index, edge_attr, pe, rwse, batch, params,
           interpret=False):
    del pe, batch
    p = params
    src = edge_index[0].astype(jnp.int32)
    dst = edge_index[1].astype(jnp.int32)
    pad = jnp.full((E2 - E,), DN, jnp.int32)
    srcp = jnp.concatenate([src, pad])
    dstp = jnp.concatenate([dst, pad])
    a_pad = jnp.concatenate([edge_attr.reshape(E).astype(jnp.float32),
                             jnp.zeros((E2 - E,), jnp.float32)]).reshape(E2, 1)
    a2d = edge_attr.reshape(E // D, D).astype(jnp.float32)

    enc_args = (
        _row(p["enc_node"]["W"][:, 0]), _row(p["enc_node"]["b"]),
        _row(p["enc_node_bn"]["g"]), _row(p["enc_node_bn"]["b"]),
        p["enc_rwse"]["W"].T, _row(p["enc_rwse"]["b"]),
        _row(p["enc_rwse_bn"]["g"]), _row(p["enc_rwse_bn"]["b"]),
        p["pre"]["W"].T, _row(p["pre"]["b"]),
        _row(p["pre_bn"]["g"]), _row(p["pre_bn"]["b"]),
        _row(p["enc_edge"]["W"][:, 0]), _row(p["enc_edge"]["b"]),
        _row(p["enc_edge_bn"]["g"]), _row(p["enc_edge_bn"]["b"]),
        p["layers"][0]["e1"]["W"][:, :D].T, _row(p["layers"][0]["e1"]["b"]),
        p["layers"][0]["e1"]["W"][:, D:2 * D].T,
    )
    h, A, B, uece = _enc_call(x, rwse, a2d, enc_args, interpret=interpret)

    weT_all = jnp.concatenate(
        [lp["e1"]["W"][:, 2 * D:].T for lp in p["layers"]], axis=1)
    cs = _cmat_call(a_pad, uece, weT_all, interpret=interpret)

    edge_k = _make_edge_kernel(interpret=interpret)
    deg_k = _make_deg_kernel(interpret=interpret)
    (dg2,) = deg_k(dstp)

    for li, lp in enumerate(p["layers"]):
        (s2,) = edge_k(A, B, cs[li], srcp, dstp)
        has_next = li + 1 < NL
        gcn_args = [
            lp["e2"]["W"].T, _row(lp["e2"]["b"]),
            lp["n1"]["W"][:, :D].T, lp["n1"]["W"][:, D:].T, _row(lp["n1"]["b"]),
            lp["n2"]["W"].T, _row(lp["n2"]["b"]),
            _row(lp["gcn_ln"]["g"]), _row(lp["gcn_ln"]["b"]),
            _row(lp["ln_local"]["g"]), _row(lp["ln_local"]["b"]),
        ]
        per_head = lambda w: w.T.reshape(D, HEADS, DH).transpose(1, 0, 2)
        projT = p["proj"].T
        merge_args = [
            lp["f1"]["W"].T, _row(lp["f1"]["b"]),
            lp["f2"]["W"].T, _row(lp["f2"]["b"]),
            _row(lp["ln_ffn"]["g"]), _row(lp["ln_ffn"]["b"]),
        ]
        hl = _gcn_call(h, s2, dg2, gcn_args, interpret=interpret)
        kmax = _kmax_call(h, lp["k"]["W"].T, _row(lp["k"]["b"]), projT,
                          interpret=interpret)
        ha = _attn_call(
            h,
            per_head(lp["q"]["W"]), lp["q"]["b"].reshape(HEADS, 1, DH),
            per_head(lp["k"]["W"]), lp["k"]["b"].reshape(HEADS, 1, DH),
            per_head(lp["v"]["W"]), lp["v"]["b"].reshape(HEADS, 1, DH),
            lp["o"]["W"].T.reshape(HEADS, DH, D),
            _row(lp["o"]["b"]), projT, kmax,
            _row(lp["ln_attn"]["g"]), _row(lp["ln_attn"]["b"]),
            interpret=interpret)
        if has_next:
            nxt = p["layers"][li + 1]
            merge_args += [nxt["e1"]["W"][:, :D].T, _row(nxt["e1"]["b"]),
                           nxt["e1"]["W"][:, D:2 * D].T]
            h, A, B = _merge_call(hl, ha, merge_args, True, interpret=interpret)
        else:
            (h,) = _merge_call(hl, ha, merge_args, False, interpret=interpret)

    return _head_call(h, p["head1"]["W"].T, _row(p["head1"]["b"]),
                      p["head2"]["W"].T, _row(p["head2"]["b"]),
                      interpret=interpret)


# speedup vs baseline: 1.1083x; 1.1083x over previous
import functools

import jax
import jax.numpy as jnp
from jax import lax
from jax.experimental import pallas as pl
from jax.experimental.pallas import tpu as pltpu
from jax.experimental.pallas import tpu_sc as plsc

N = 10000
E = 160000
D = 128
HEADS = 8
DH = 16
M = 64
NL = 6

NP = 10112
DN = N
NW = 32
K = 48
NB = 106
EW = NB * K
E2 = NW * EW
RPS = NP // 16

_EPS = 1e-5


def _ln(x, g, b):
    m = x.mean(-1, keepdims=True)
    d = x - m
    v = (d * d).mean(-1, keepdims=True)
    return d / jnp.sqrt(v + _EPS) * g + b


def _enc_body(x_ref, rwse_ref, a2d_ref,
              wn_ref, bn_ref, gn_ref, zn_ref,
              wrT_ref, br_ref, gr_ref, zr_ref,
              wpT_ref, bp_ref, gp_ref, zp_ref,
              we_ref, be_ref, ge_ref, ze_ref,
              w1iT_ref, b1_ref, w1jT_ref,
              h0_ref, a0_ref, b0_ref, uece_ref):
    x = x_ref[...]
    am = jnp.mean(x)
    dx = x - am
    av = jnp.mean(dx * dx)
    w = wn_ref[...]
    inv = gn_ref[...] / jnp.sqrt(av * w * w + _EPS)
    h = jnp.maximum(dx * (w * inv) + zn_ref[...], 0.0)
    z = jnp.dot(rwse_ref[...], wrT_ref[...],
                preferred_element_type=jnp.float32) + br_ref[...]
    zm = z.mean(0, keepdims=True)
    dz = z - zm
    zv = (dz * dz).mean(0, keepdims=True)
    h = h + jnp.maximum(dz / jnp.sqrt(zv + _EPS) * gr_ref[...] + zr_ref[...], 0.0)
    z = jnp.maximum(jnp.dot(h, wpT_ref[...],
                            preferred_element_type=jnp.float32) + bp_ref[...], 0.0)
    zm = z.mean(0, keepdims=True)
    dz = z - zm
    zv = (dz * dz).mean(0, keepdims=True)
    h0 = dz / jnp.sqrt(zv + _EPS) * gp_ref[...] + zp_ref[...]
    h0_ref[...] = h0
    a2 = a2d_ref[...]
    em = jnp.mean(a2)
    de = a2 - em
    ev = jnp.mean(de * de)
    we = we_ref[...]
    inve = ge_ref[...] / jnp.sqrt(ev * we * we + _EPS)
    uece_ref[0:1, :] = we * inve
    uece_ref[1:2, :] = (-em * we) * inve + ze_ref[...]
    a0_ref[pl.ds(0, N), :] = jnp.dot(h0, w1iT_ref[...],
                                     preferred_element_type=jnp.float32) + b1_ref[...]
    a0_ref[pl.ds(N, NP - N), :] = jnp.zeros((NP - N, D), jnp.float32)
    b0_ref[pl.ds(0, N), :] = jnp.dot(h0, w1jT_ref[...],
                                     preferred_element_type=jnp.float32)
    b0_ref[pl.ds(N, NP - N), :] = jnp.zeros((NP - N, D), jnp.float32)


def _enc_call(x, rwse, a2d, args, interpret=False):
    return pl.pallas_call(
        _enc_body,
        out_shape=[
            jax.ShapeDtypeStruct((N, D), jnp.float32),
            jax.ShapeDtypeStruct((NP, D), jnp.float32),
            jax.ShapeDtypeStruct((NP, D), jnp.float32),
            jax.ShapeDtypeStruct((2, D), jnp.float32),
        ],
        interpret=interpret,
    )(x, rwse, a2d, *args)


_CB = 1536


def _cmat_body(a_ref, uece_ref, weT_ref, *out_refs):
    a = a_ref[...]
    ue = uece_ref[0:1, :]
    ce = uece_ref[1:2, :]
    ea = jnp.maximum(a * ue + ce, 0.0)
    for l in range(NL):
        out_refs[l][...] = jnp.dot(ea, weT_ref[:, l * D:(l + 1) * D],
                                   preferred_element_type=jnp.float32)


def _cmat_call(a_pad, uece, weT_all, interpret=False):
    grid = E2 // _CB
    return pl.pallas_call(
        _cmat_body,
        grid=(grid,),
        in_specs=[
            pl.BlockSpec((_CB, 1), lambda i: (i, 0)),
            pl.BlockSpec((2, D), lambda i: (0, 0)),
            pl.BlockSpec((D, NL * D), lambda i: (0, 0)),
        ],
        out_specs=[pl.BlockSpec((_CB, D), lambda i: (i, 0)) for _ in range(NL)],
        out_shape=[jax.ShapeDtypeStruct((E2, D), jnp.float32) for _ in range(NL)],
        interpret=interpret,
    )(a_pad, uece, weT_all)


def _make_edge_kernel(interpret=False):
    mesh = plsc.VectorSubcoreMesh(core_axis_name="c", subcore_axis_name="s",
                                  num_cores=2, num_subcores=16)
    out_type = [jax.ShapeDtypeStruct((2, NP, D), jnp.float32)]
    scratch = [
        [pltpu.VMEM((K,), jnp.int32) for _ in range(2)],
        [pltpu.VMEM((K,), jnp.int32) for _ in range(2)],
        [pltpu.VMEM((K,), jnp.int32) for _ in range(2)],
        [pltpu.VMEM((K, D), jnp.float32) for _ in range(2)],
        [pltpu.VMEM((K, D), jnp.float32) for _ in range(2)],
        [pltpu.VMEM((K, D), jnp.float32) for _ in range(2)],
        [pltpu.VMEM((K, D), jnp.float32) for _ in range(2)],
        pltpu.VMEM_SHARED((NP, D), jnp.float32),
        [pltpu.SemaphoreType.DMA for _ in range(2)],
        [pltpu.SemaphoreType.DMA for _ in range(2)],
        [pltpu.SemaphoreType.DMA for _ in range(2)],
        [pltpu.SemaphoreType.DMA for _ in range(2)],
    ]

    def body(a_hbm, b_hbm, c_hbm, src_hbm, dst_hbm, out_s,
             idx_s, idx_d, idx_sc, buf_a, buf_b, buf_c, buf_z, s_sh,
             sem_a, sem_b, sem_c, sem_sc):
        cid = lax.axis_index("c")
        sid = lax.axis_index("s")
        wid = sid * 2 + cid
        base0 = wid * EW

        def zrow(r, carry):
            for j in range(D // 16):
                buf_z[0][r, pl.ds(j * 16, 16)] = jnp.zeros((16,), jnp.float32)
            return carry
        lax.fori_loop(0, K, zrow, 0)
        r0 = sid * RPS
        off = 0
        while off < RPS:
            sz = min(K, RPS - off)
            pltpu.sync_copy(buf_z[0].at[pl.ds(0, sz)],
                            s_sh.at[pl.ds(r0 + off, sz)])
            off += sz
        plsc.subcore_barrier()

        def start_gathers(q, b):
            base = base0 + b * K
            pltpu.sync_copy(src_hbm.at[pl.ds(base, K)], idx_s[q])
            pltpu.sync_copy(dst_hbm.at[pl.ds(base, K)], idx_d[q])
            pltpu.async_copy(a_hbm.at[idx_d[q]], buf_a[q], sem_a[q])
            pltpu.async_copy(b_hbm.at[idx_s[q]], buf_b[q], sem_b[q])
            pltpu.async_copy(c_hbm.at[pl.ds(base, K)], buf_c[q], sem_c[q])

        def wait_gathers(q):
            pltpu.make_async_copy(a_hbm.at[idx_d[q]], buf_a[q], sem_a[q]).wait()
            pltpu.make_async_copy(b_hbm.at[idx_s[q]], buf_b[q], sem_b[q]).wait()
            pltpu.make_async_copy(c_hbm.at[pl.ds(0, K)], buf_c[q], sem_c[q]).wait()

        for q in (0, 1):
            start_gathers(q, q)

        def step(i, carry):
            for q in (0, 1):
                b = 2 * i + q
                wait_gathers(q)

                @pl.when(i >= 1)
                def _drain():
                    pltpu.make_async_copy(
                        buf_z[q], s_sh.at[idx_sc[q]], sem_sc[q]).wait()

                def crow(r, carry2):
                    for j in range(D // 16):
                        s = pl.ds(j * 16, 16)
                        buf_z[q][r, s] = jnp.maximum(
                            buf_a[q][r, s] + buf_b[q][r, s] + buf_c[q][r, s],
                            0.0)
                    return carry2
                lax.fori_loop(0, K, crow, 0)
                for j in range(K // 16):
                    s = pl.ds(j * 16, 16)
                    idx_sc[q][s] = idx_d[q][s]
                pltpu.async_copy(buf_z[q], s_sh.at[idx_sc[q]], sem_sc[q],
                                 add=True)

                @pl.when(b + 2 < NB)
                def _prefetch():
                    start_gathers(q, b + 2)
            return carry
        lax.fori_loop(0, NB // 2, step, 0)
        for q in (0, 1):
            pltpu.make_async_copy(buf_z[q], s_sh.at[idx_sc[q]], sem_sc[q]).wait()
        plsc.subcore_barrier()

        pltpu.sync_copy(s_sh.at[pl.ds(r0, RPS)], out_s.at[cid, pl.ds(r0, RPS)])

    return pl.kernel(body, out_type=out_type, mesh=mesh, scratch_types=scratch,
                     interpret=interpret)


def _make_deg_kernel(interpret=False):
    mesh = plsc.VectorSubcoreMesh(core_axis_name="c", subcore_axis_name="s",
                                  num_cores=2, num_subcores=16)
    out_type = [jax.ShapeDtypeStruct((2, NP, 16), jnp.float32)]
    scratch = [
        pltpu.VMEM((K,), jnp.int32),
        pltpu.VMEM((K, 16), jnp.float32),
        pltpu.VMEM((K, 16), jnp.float32),
        pltpu.VMEM_SHARED((NP, 16), jnp.float32),
        pltpu.SemaphoreType.DMA,
    ]

    def body(dst_hbm, out_d, idx_d, ones16, zer16, d_sh, sem):
        del sem
        cid = lax.axis_index("c")
        sid = lax.axis_index("s")
        wid = sid * 2 + cid

        def frow(r, carry):
            ones16[r, pl.ds(0, 16)] = jnp.ones((16,), jnp.float32)
            zer16[r, pl.ds(0, 16)] = jnp.zeros((16,), jnp.float32)
            return carry
        lax.fori_loop(0, K, frow, 0)
        r0 = sid * RPS
        off = 0
        while off < RPS:
            sz = min(K, RPS - off)
            pltpu.sync_copy(zer16.at[pl.ds(0, sz)], d_sh.at[pl.ds(r0 + off, sz)])
            off += sz
        plsc.subcore_barrier()

        def blk(b, carry):
            base = wid * EW + b * K
            pltpu.sync_copy(dst_hbm.at[pl.ds(base, K)], idx_d)
            pltpu.sync_copy(ones16, d_sh.at[idx_d], add=True)
            return carry
        lax.fori_loop(0, NB, blk, 0)
        plsc.subcore_barrier()
        pltpu.sync_copy(d_sh.at[pl.ds(r0, RPS)], out_d.at[cid, pl.ds(r0, RPS)])

    return pl.kernel(body, out_type=out_type, mesh=mesh, scratch_types=scratch,
                     interpret=interpret)


_DN_SCALE = DH ** -0.25
_RATIO = M ** -0.5


def _gcn_body(h_ref, s2_ref, dg2_ref,
              e2T_ref, b2_ref, n1hT_ref, n1aT_ref, bn1_ref, n2T_ref, bn2_ref,
              ggcn_ref, zgcn_ref, gloc_ref, zloc_ref, hl_ref):
    h = h_ref[...]
    su = s2_ref[0, pl.ds(0, N), :] + s2_ref[1, pl.ds(0, N), :]
    dgf = dg2_ref[0, pl.ds(0, N), :] + dg2_ref[1, pl.ds(0, N), :]
    dg = dgf[:, 0:1]
    agg = jnp.dot(su, e2T_ref[...], preferred_element_type=jnp.float32) \
        + dg * b2_ref[...]
    t = jnp.maximum(
        jnp.dot(h, n1hT_ref[...], preferred_element_type=jnp.float32)
        + jnp.dot(agg, n1aT_ref[...], preferred_element_type=jnp.float32)
        + bn1_ref[...], 0.0)
    og = jnp.dot(t, n2T_ref[...], preferred_element_type=jnp.float32) \
        + bn2_ref[...] + h
    hl = _ln(og, ggcn_ref[...], zgcn_ref[...])
    hl_ref[...] = _ln(h + hl, gloc_ref[...], zloc_ref[...])


def _gcn_call(h, s2, dg2, args, interpret=False):
    return pl.pallas_call(
        _gcn_body,
        out_shape=jax.ShapeDtypeStruct((N, D), jnp.float32),
        interpret=interpret,
    )(h, s2, dg2, *args)


def _kmax_body(h_ref, kT_ref, kb_ref, projT_ref, out_ref):
    h = h_ref[...]
    kT = kT_ref[...]
    kb = kb_ref[...]
    projT = projT_ref[...]
    vals = []
    for hh in range(HEADS):
        sl = slice(hh * DH, (hh + 1) * DH)
        kh = (jnp.dot(h, kT[:, sl], preferred_element_type=jnp.float32)
              + kb[:, sl]) * _DN_SCALE
        ukh = jnp.dot(kh, projT, preferred_element_type=jnp.float32)
        vals.append(jnp.max(ukh).reshape(1, 1))
    out_ref[...] = jnp.concatenate(vals, axis=1)


def _kmax_call(h, kT, kb, projT, interpret=False):
    return pl.pallas_call(
        _kmax_body,
        out_shape=jax.ShapeDtypeStruct((1, HEADS), jnp.float32),
        interpret=interpret,
    )(h, kT, kb, projT)


def _attn_body(h_ref, qTh_ref, qbh_ref, kTh_ref, kbh_ref, vTh_ref, vbh_ref,
               oTh_ref, ob_ref, projT_ref, kmax_ref, gat_ref, zat_ref,
               ha_ref, acc_ref):
    i = pl.program_id(0)
    h = h_ref[...]
    projT = projT_ref[...]
    mk = jnp.max(kmax_ref[...])
    qh = (jnp.dot(h, qTh_ref[0], preferred_element_type=jnp.float32)
          + qbh_ref[0]) * _DN_SCALE
    kh = (jnp.dot(h, kTh_ref[0], preferred_element_type=jnp.float32)
          + kbh_ref[0]) * _DN_SCALE
    vh = jnp.dot(h, vTh_ref[0], preferred_element_type=jnp.float32) + vbh_ref[0]
    uq = jnp.dot(qh, projT, preferred_element_type=jnp.float32)
    uk = jnp.dot(kh, projT, preferred_element_type=jnp.float32)
    d2q = 0.5 * jnp.sum(qh * qh, axis=1, keepdims=True)
    d2k = 0.5 * jnp.sum(kh * kh, axis=1, keepdims=True)
    mq = jnp.max(uq, axis=1, keepdims=True)
    qp = _RATIO * (jnp.exp(uq - d2q - mq) + 1e-4)
    kp = _RATIO * (jnp.exp(uk - d2k - mk) + 1e-4)
    vh1 = jnp.concatenate([vh, jnp.ones((N, 1), jnp.float32)], axis=1)
    ctx = lax.dot_general(kp, vh1, (((0,), (0,)), ((), ())),
                          preferred_element_type=jnp.float32)
    num = jnp.dot(qp, ctx, preferred_element_type=jnp.float32)
    o_h = num[:, :DH] / (num[:, DH:DH + 1] + 1e-6)
    contrib = jnp.dot(o_h, oTh_ref[0], preferred_element_type=jnp.float32)

    @pl.when(i == 0)
    def _init():
        acc_ref[...] = contrib

    @pl.when(i > 0)
    def _acc():
        acc_ref[...] = acc_ref[...] + contrib

    @pl.when(i == HEADS - 1)
    def _fin():
        attn = acc_ref[...] + ob_ref[...]
        ha_ref[...] = _ln(h + attn, gat_ref[...], zat_ref[...])


def _attn_call(h, qTh, qbh, kTh, kbh, vTh, vbh, oTh, ob, projT, kmax,
               gat, zat, interpret=False):
    full2 = lambda shape: pl.BlockSpec(shape, lambda i: (0, 0))
    headb = lambda shape: pl.BlockSpec(shape, lambda i: (i, 0, 0))
    return pl.pallas_call(
        _attn_body,
        grid=(HEADS,),
        in_specs=[
            full2((N, D)),
            headb((1, D, DH)), headb((1, 1, DH)),
            headb((1, D, DH)), headb((1, 1, DH)),
            headb((1, D, DH)), headb((1, 1, DH)),
            headb((1, DH, D)),
            full2((1, D)), full2((DH, M)), full2((1, HEADS)),
            full2((1, D)), full2((1, D)),
        ],
        out_specs=pl.BlockSpec((N, D), lambda i: (0, 0)),
        out_shape=jax.ShapeDtypeStruct((N, D), jnp.float32),
        scratch_shapes=[pltpu.VMEM((N, D), jnp.float32)],
        compiler_params=pltpu.CompilerParams(vmem_limit_bytes=100 * 1024 * 1024),
        interpret=interpret,
    )(h, qTh, qbh, kTh, kbh, vTh, vbh, oTh, ob, projT, kmax, gat, zat)


def _merge_body(has_next, *refs):
    (hl_ref, ha_ref, f1T_ref, f1b_ref, f2T_ref, f2b_ref,
     gff_ref, zff_ref) = refs[:8]
    if has_next:
        (w1iT_ref, b1n_ref, w1jT_ref) = refs[8:11]
        (hout_ref, an_ref, bn_ref) = refs[11:]
    else:
        (hout_ref,) = refs[8:]
    hn = hl_ref[...] + ha_ref[...]
    ff = jnp.dot(
        jnp.maximum(jnp.dot(hn, f1T_ref[...],
                            preferred_element_type=jnp.float32) + f1b_ref[...], 0.0),
        f2T_ref[...], preferred_element_type=jnp.float32) + f2b_ref[...]
    hout = _ln(hn + ff, gff_ref[...], zff_ref[...])
    hout_ref[...] = hout
    if has_next:
        an_ref[pl.ds(0, N), :] = jnp.dot(
            hout, w1iT_ref[...], preferred_element_type=jnp.float32) + b1n_ref[...]
        an_ref[pl.ds(N, NP - N), :] = jnp.zeros((NP - N, D), jnp.float32)
        bn_ref[pl.ds(0, N), :] = jnp.dot(
            hout, w1jT_ref[...], preferred_element_type=jnp.float32)
        bn_ref[pl.ds(N, NP - N), :] = jnp.zeros((NP - N, D), jnp.float32)


def _merge_call(hl, ha, args, has_next, interpret=False):
    out_shape = [jax.ShapeDtypeStruct((N, D), jnp.float32)]
    if has_next:
        out_shape.append(jax.ShapeDtypeStruct((NP, D), jnp.float32))
        out_shape.append(jax.ShapeDtypeStruct((NP, D), jnp.float32))
    return pl.pallas_call(
        functools.partial(_merge_body, has_next),
        out_shape=out_shape,
        interpret=interpret,
    )(hl, ha, *args)


def _head_body(h_ref, w1T_ref, b1_ref, w2T_ref, b2_ref, out_ref):
    t = jnp.maximum(jnp.dot(h_ref[...], w1T_ref[...],
                            preferred_element_type=jnp.float32) + b1_ref[...], 0.0)
    out_ref[...] = jnp.dot(t, w2T_ref[...],
                           preferred_element_type=jnp.float32) + b2_ref[...]


def _head_call(h, w1T, b1, w2T, b2, interpret=False):
    return pl.pallas_call(
        _head_body,
        out_shape=jax.ShapeDtypeStruct((N, 1), jnp.float32),
        interpret=interpret,
    )(h, w1T, b1, w2T, b2)


def _row(x):
    return x.reshape(1, -1).astype(jnp.float32)


def kernel(x, edge_index, edge_attr, pe, rwse, batch, params,
           interpret=False):
    del pe, batch
    p = params
    src = edge_index[0].astype(jnp.int32)
    dst = edge_index[1].astype(jnp.int32)
    pad = jnp.full((E2 - E,), DN, jnp.int32)
    srcp = jnp.concatenate([src, pad])
    dstp = jnp.concatenate([dst, pad])
    a_pad = jnp.concatenate([edge_attr.reshape(E).astype(jnp.float32),
                             jnp.zeros((E2 - E,), jnp.float32)]).reshape(E2, 1)
    a2d = edge_attr.reshape(E // D, D).astype(jnp.float32)

    enc_args = (
        _row(p["enc_node"]["W"][:, 0]), _row(p["enc_node"]["b"]),
        _row(p["enc_node_bn"]["g"]), _row(p["enc_node_bn"]["b"]),
        p["enc_rwse"]["W"].T, _row(p["enc_rwse"]["b"]),
        _row(p["enc_rwse_bn"]["g"]), _row(p["enc_rwse_bn"]["b"]),
        p["pre"]["W"].T, _row(p["pre"]["b"]),
        _row(p["pre_bn"]["g"]), _row(p["pre_bn"]["b"]),
        _row(p["enc_edge"]["W"][:, 0]), _row(p["enc_edge"]["b"]),
        _row(p["enc_edge_bn"]["g"]), _row(p["enc_edge_bn"]["b"]),
        p["layers"][0]["e1"]["W"][:, :D].T, _row(p["layers"][0]["e1"]["b"]),
        p["layers"][0]["e1"]["W"][:, D:2 * D].T,
    )
    h, A, B, uece = _enc_call(x, rwse, a2d, enc_args, interpret=interpret)

    weT_all = jnp.concatenate(
        [lp["e1"]["W"][:, 2 * D:].T for lp in p["layers"]], axis=1)
    cs = _cmat_call(a_pad, uece, weT_all, interpret=interpret)

    edge_k = _make_edge_kernel(interpret=interpret)
    deg_k = _make_deg_kernel(interpret=interpret)
    (dg2,) = deg_k(dstp)

    for li, lp in enumerate(p["layers"]):
        (s2,) = edge_k(A, B, cs[li], srcp, dstp)
        has_next = li + 1 < NL
        gcn_args = [
            lp["e2"]["W"].T, _row(lp["e2"]["b"]),
            lp["n1"]["W"][:, :D].T, lp["n1"]["W"][:, D:].T, _row(lp["n1"]["b"]),
            lp["n2"]["W"].T, _row(lp["n2"]["b"]),
            _row(lp["gcn_ln"]["g"]), _row(lp["gcn_ln"]["b"]),
            _row(lp["ln_local"]["g"]), _row(lp["ln_local"]["b"]),
        ]
        per_head = lambda w: w.T.reshape(D, HEADS, DH).transpose(1, 0, 2)
        projT = p["proj"].T
        merge_args = [
            lp["f1"]["W"].T, _row(lp["f1"]["b"]),
            lp["f2"]["W"].T, _row(lp["f2"]["b"]),
            _row(lp["ln_ffn"]["g"]), _row(lp["ln_ffn"]["b"]),
        ]
        hl = _gcn_call(h, s2, dg2, gcn_args, interpret=interpret)
        kmax = _kmax_call(h, lp["k"]["W"].T, _row(lp["k"]["b"]), projT,
                          interpret=interpret)
        ha = _attn_call(
            h,
            per_head(lp["q"]["W"]), lp["q"]["b"].reshape(HEADS, 1, DH),
            per_head(lp["k"]["W"]), lp["k"]["b"].reshape(HEADS, 1, DH),
            per_head(lp["v"]["W"]), lp["v"]["b"].reshape(HEADS, 1, DH),
            lp["o"]["W"].T.reshape(HEADS, DH, D),
            _row(lp["o"]["b"]), projT, kmax,
            _row(lp["ln_attn"]["g"]), _row(lp["ln_attn"]["b"]),
            interpret=interpret)
        if has_next:
            nxt = p["layers"][li + 1]
            merge_args += [nxt["e1"]["W"][:, :D].T, _row(nxt["e1"]["b"]),
                           nxt["e1"]["W"][:, D:2 * D].T]
            h, A, B = _merge_call(hl, ha, merge_args, True, interpret=interpret)
        else:
            (h,) = _merge_call(hl, ha, merge_args, False, interpret=interpret)

    return _head_call(h, p["head1"]["W"].T, _row(p["head1"]["b"]),
                      p["head2"]["W"].T, _row(p["head2"]["b"]),
                      interpret=interpret)
